# Initial kernel scaffold; baseline (speedup 1.0000x reference)
import functools

import jax
import jax.numpy as jnp
from jax import lax
from jax.experimental import pallas as pl
from jax.experimental.pallas import tpu as pltpu
from jax.experimental.pallas import tpu_sc as plsc

N = 10000
E = 320000
D = 128

NC = 2
NS = 16
NW = NC * NS

K = 128
NCHUNK = -(-E // (NW * K))
EPW = NCHUNK * K
EPAD = NW * EPW
TRASH = N
HPAD = ((N + 1 + NS - 1) // NS + 1) * NS
RPT = HPAD // NS

_mesh = plsc.VectorSubcoreMesh(
    core_axis_name="c", subcore_axis_name="s", num_cores=NC, num_subcores=NS
)


@functools.partial(
    pl.kernel,
    out_type=jax.ShapeDtypeStruct((NW, HPAD), jnp.float32),
    mesh=_mesh,
    scratch_types=[
        pltpu.VMEM((EPW,), jnp.int32),
        pltpu.VMEM((HPAD,), jnp.float32),
    ],
)
def _deg_kernel(dst_hbm, out_hbm, idx_v, hist_v):
    wid = lax.axis_index("s") * NC + lax.axis_index("c")
    zero16 = jnp.zeros((16,), jnp.float32)

    def zbody(i, carry):
        hist_v[pl.ds(i * 16, 16)] = zero16
        return carry

    lax.fori_loop(0, HPAD // 16, zbody, 0)
    pltpu.sync_copy(dst_hbm.at[pl.ds(wid * EPW, EPW)], idx_v)
    ones16 = jnp.ones((16,), jnp.float32)

    def body(i, carry):
        idx = idx_v[pl.ds(i * 16, 16)]
        plsc.addupdate_scatter(hist_v, [idx], ones16)
        return carry

    lax.fori_loop(0, EPW // 16, body, 0)
    pltpu.sync_copy(hist_v, out_hbm.at[wid])


def _mm_body(parts_ref, x_ref, w_ref, g_ref, dis_ref):
    deg = jnp.sum(parts_ref[...], axis=0) + 1.0
    dis = lax.rsqrt(deg)
    h = jnp.dot(x_ref[...], w_ref[...], preferred_element_type=jnp.float32)
    g_ref[...] = dis[:, None] * h
    dis_ref[...] = dis[:, None]


_R = 2000


def _mm_call(parts, x, W):
    return pl.pallas_call(
        _mm_body,
        grid=(N // _R,),
        in_specs=[
            pl.BlockSpec((NW, _R), lambda i: (0, i)),
            pl.BlockSpec((_R, D), lambda i: (i, 0)),
            pl.BlockSpec((D, D), lambda i: (0, 0)),
        ],
        out_specs=[
            pl.BlockSpec((_R, D), lambda i: (i, 0)),
            pl.BlockSpec((_R, 1), lambda i: (i, 0)),
        ],
        out_shape=[
            jax.ShapeDtypeStruct((N, D), jnp.float32),
            jax.ShapeDtypeStruct((N, 1), jnp.float32),
        ],
    )(parts, x, W)


@functools.partial(
    pl.kernel,
    out_type=jax.ShapeDtypeStruct((NC, HPAD, D), jnp.float32),
    mesh=_mesh,
    scratch_types=[
        pltpu.VMEM((K,), jnp.int32),
        pltpu.VMEM((K,), jnp.int32),
        pltpu.VMEM((K, D), jnp.float32),
        pltpu.VMEM_SHARED((HPAD, D), jnp.float32),
        pltpu.SemaphoreType.DMA,
    ],
)
def _edge_kernel(g_hbm, src_hbm, dst_hbm, zeros_hbm, out_hbm,
                 src_v, dst_v, rows_v, acc_sh, sem):
    cid = lax.axis_index("c")
    sid = lax.axis_index("s")
    wid = sid * NC + cid

    pltpu.sync_copy(
        zeros_hbm.at[pl.ds(sid * RPT, RPT)], acc_sh.at[pl.ds(sid * RPT, RPT)]
    )
    plsc.subcore_barrier()

    def body(i, carry):
        base = (wid * NCHUNK + i) * K
        pltpu.sync_copy(src_hbm.at[pl.ds(base, K)], src_v)
        pltpu.sync_copy(dst_hbm.at[pl.ds(base, K)], dst_v)
        pltpu.async_copy(g_hbm.at[src_v], rows_v, sem).wait()
        pltpu.sync_copy(rows_v, acc_sh.at[dst_v], add=True)
        return carry

    lax.fori_loop(0, NCHUNK, body, 0)
    plsc.subcore_barrier()
    pltpu.sync_copy(
        acc_sh.at[pl.ds(sid * RPT, RPT)],
        out_hbm.at[cid, pl.ds(sid * RPT, RPT)],
    )


def _ep_body(p_ref, g_ref, dis_ref, b_ref, o_ref):
    s = p_ref[0] + p_ref[1] + g_ref[...]
    y = dis_ref[...] * s + b_ref[...]
    o_ref[...] = jnp.where(y >= 0, y, 0.01 * y)


def _ep_call(partial, g, dis, b2):
    return pl.pallas_call(
        _ep_body,
        grid=(N // _R,),
        in_specs=[
            pl.BlockSpec((NC, _R, D), lambda i: (0, i, 0)),
            pl.BlockSpec((_R, D), lambda i: (i, 0)),
            pl.BlockSpec((_R, 1), lambda i: (i, 0)),
            pl.BlockSpec((1, D), lambda i: (0, 0)),
        ],
        out_specs=pl.BlockSpec((_R, D), lambda i: (i, 0)),
        out_shape=jax.ShapeDtypeStruct((N, D), jnp.float32),
    )(partial, g, dis, b2)


def kernel(x, edge_index, W, b):
    src = edge_index[0].astype(jnp.int32)
    dst = edge_index[1].astype(jnp.int32)
    pad = EPAD - E
    src_p = jnp.concatenate([src, jnp.zeros((pad,), jnp.int32)])
    dst_p = jnp.concatenate([dst, jnp.full((pad,), TRASH, jnp.int32)])

    parts = _deg_kernel(dst_p)
    g, dis = _mm_call(parts, x, W)
    zeros = jnp.zeros((HPAD, D), jnp.float32)
    partial = _edge_kernel(g, src_p, dst_p, zeros)
    return _ep_call(partial, g, dis, b.reshape(1, D))


# speedup vs baseline: 17.8117x; 17.8117x over previous
import functools

import jax
import jax.numpy as jnp
from jax import lax
from jax.experimental import pallas as pl
from jax.experimental.pallas import tpu as pltpu
from jax.experimental.pallas import tpu_sc as plsc

N = 10000
E = 320000
D = 128

NC = 2
NS = 16
NW = NC * NS

K = 128
NCHUNK = -(-E // (NW * K))
EPW = NCHUNK * K
EPAD = NW * EPW
TRASH = N
HPAD = ((N + 1 + NS * 8 - 1) // (NS * 8)) * NS * 8
RPT = HPAD // NS

_mesh = plsc.VectorSubcoreMesh(
    core_axis_name="c", subcore_axis_name="s", num_cores=NC, num_subcores=NS
)


@functools.partial(
    pl.kernel,
    out_type=jax.ShapeDtypeStruct((NW, HPAD), jnp.float32),
    mesh=_mesh,
    scratch_types=[
        pltpu.VMEM((EPW,), jnp.int32),
        pltpu.VMEM((HPAD,), jnp.float32),
    ],
    compiler_params=pltpu.CompilerParams(needs_layout_passes=False),
)
def _deg_kernel(dst_hbm, out_hbm, idx_v, hist_v):
    wid = lax.axis_index("s") * NC + lax.axis_index("c")
    zero16 = jnp.zeros((16,), jnp.float32)

    def zbody(i, carry):
        hist_v[pl.ds(i * 16, 16)] = zero16
        return carry

    lax.fori_loop(0, HPAD // 16, zbody, 0)
    pltpu.sync_copy(dst_hbm.at[pl.ds(wid * EPW, EPW)], idx_v)
    ones16 = jnp.ones((16,), jnp.float32)

    def body(i, carry):
        idx = idx_v[pl.ds(i * 16, 16)]
        plsc.addupdate_scatter(hist_v, [idx], ones16)
        return carry

    lax.fori_loop(0, EPW // 16, body, 0)
    pltpu.sync_copy(hist_v, out_hbm.at[wid])


def _mm_body(parts_ref, x_ref, w_ref, g_ref, dis_ref):
    deg = jnp.sum(parts_ref[...], axis=1) + 1.0
    dis = lax.rsqrt(deg)
    h = jnp.dot(x_ref[...], w_ref[...], preferred_element_type=jnp.float32)
    g_ref[...] = dis[:, None] * h
    dis_ref[...] = dis[:, None]


_R = 2000


def _mm_call(parts, x, W):
    return pl.pallas_call(
        _mm_body,
        grid=(N // _R,),
        in_specs=[
            pl.BlockSpec((_R, NW), lambda i: (i, 0)),
            pl.BlockSpec((_R, D), lambda i: (i, 0)),
            pl.BlockSpec((D, D), lambda i: (0, 0)),
        ],
        out_specs=[
            pl.BlockSpec((_R, D), lambda i: (i, 0)),
            pl.BlockSpec((_R, 1), lambda i: (i, 0)),
        ],
        out_shape=[
            jax.ShapeDtypeStruct((N, D), jnp.float32),
            jax.ShapeDtypeStruct((N, 1), jnp.float32),
        ],
    )(parts, x, W)


@functools.partial(
    pl.kernel,
    out_type=jax.ShapeDtypeStruct((NC, HPAD, D), jnp.float32),
    mesh=_mesh,
    scratch_types=[
        pltpu.VMEM((K,), jnp.int32),
        pltpu.VMEM((K,), jnp.int32),
        pltpu.VMEM((K, D), jnp.float32),
        pltpu.VMEM_SHARED((HPAD, D), jnp.float32),
        pltpu.SemaphoreType.DMA,
    ],
)
def _edge_kernel(g_hbm, src_hbm, dst_hbm, zeros_hbm, out_hbm,
                 src_v, dst_v, rows_v, acc_sh, sem):
    cid = lax.axis_index("c")
    sid = lax.axis_index("s")
    wid = sid * NC + cid

    pltpu.sync_copy(
        zeros_hbm.at[pl.ds(sid * RPT, RPT)], acc_sh.at[pl.ds(sid * RPT, RPT)]
    )
    plsc.subcore_barrier()

    def body(i, carry):
        base = (wid * NCHUNK + i) * K
        pltpu.sync_copy(src_hbm.at[pl.ds(base, K)], src_v)
        pltpu.sync_copy(dst_hbm.at[pl.ds(base, K)], dst_v)
        pltpu.async_copy(g_hbm.at[src_v], rows_v, sem).wait()
        pltpu.sync_copy(rows_v, acc_sh.at[dst_v], add=True)
        return carry

    lax.fori_loop(0, NCHUNK, body, 0)
    plsc.subcore_barrier()
    pltpu.sync_copy(
        acc_sh.at[pl.ds(sid * RPT, RPT)],
        out_hbm.at[cid, pl.ds(sid * RPT, RPT)],
    )


def _ep_body(p_ref, g_ref, dis_ref, b_ref, o_ref):
    s = p_ref[0] + p_ref[1] + g_ref[...]
    y = dis_ref[...] * s + b_ref[...]
    o_ref[...] = jnp.where(y >= 0, y, 0.01 * y)


def _ep_call(partial, g, dis, b2):
    return pl.pallas_call(
        _ep_body,
        grid=(N // _R,),
        in_specs=[
            pl.BlockSpec((NC, _R, D), lambda i: (0, i, 0)),
            pl.BlockSpec((_R, D), lambda i: (i, 0)),
            pl.BlockSpec((_R, 1), lambda i: (i, 0)),
            pl.BlockSpec((1, D), lambda i: (0, 0)),
        ],
        out_specs=pl.BlockSpec((_R, D), lambda i: (i, 0)),
        out_shape=jax.ShapeDtypeStruct((N, D), jnp.float32),
    )(partial, g, dis, b2)


def kernel(x, edge_index, W, b):
    src = edge_index[0].astype(jnp.int32)
    dst = edge_index[1].astype(jnp.int32)
    pad = EPAD - E
    src_p = jnp.concatenate([src, jnp.zeros((pad,), jnp.int32)])
    dst_p = jnp.concatenate([dst, jnp.full((pad,), TRASH, jnp.int32)])

    parts = _deg_kernel(dst_p)
    g, dis = _mm_call(parts.T, x, W)
    zeros = jnp.zeros((HPAD, D), jnp.float32)
    partial = _edge_kernel(g, src_p, dst_p, zeros)
    return _ep_call(partial, g, dis, b.reshape(1, D))
